# baseline (speedup 1.0000x reference)
import jax
import jax.numpy as jnp
from jax import lax
from jax.experimental import pallas as pl
from jax.experimental.pallas import tpu as pltpu
from jax.experimental.pallas import tpu_sc as plsc

_NUM_CORES = 2
_NUM_SUBCORES = 16
_NUM_WORKERS = _NUM_CORES * _NUM_SUBCORES
_LANES = 16

_SIZE = 8388608
_CHUNK = 16384
_NCHT = _SIZE // _CHUNK
_PER_W = _SIZE // _NUM_WORKERS
_CHUNKS_PER_W = _PER_W // _CHUNK


def _lerp_body(const2d, pair_idx, dist16, out_hbm,
               idx_v, in_buf, out_buf, dist_v, in_sem, out_sem):
    c = lax.axis_index("c")
    s = lax.axis_index("s")
    w = s * _NUM_CORES + c
    base_chunk = w * _CHUNKS_PER_W

    pltpu.sync_copy(pair_idx.at[pl.ds(base_chunk, _CHUNKS_PER_W)], idx_v)
    pltpu.sync_copy(dist16, dist_v)
    dist = dist_v[...]
    omd = 1.0 - dist

    def chunk_body(k, carry):
        pltpu.async_copy(const2d.at[idx_v.at[k]], in_buf, in_sem).wait()

        def vec_body(j, carry2):
            l = in_buf[0, pl.ds(j * _LANES, _LANES)]
            r = in_buf[1, pl.ds(j * _LANES, _LANES)]
            out_buf[pl.ds(j * _LANES, _LANES)] = l * dist + r * omd
            return carry2

        lax.fori_loop(0, _CHUNK // _LANES, vec_body, 0, unroll=8)
        pltpu.async_copy(
            out_buf, out_hbm.at[pl.ds(w * _PER_W + k * _CHUNK, _CHUNK)],
            out_sem).wait()
        return carry

    lax.fori_loop(0, _CHUNKS_PER_W, chunk_body, 0)


def kernel(lam, const, pivots):
    kernel_num = const.shape[0]
    lam_ = lam * 0.99999
    left = jnp.floor(lam_ * (kernel_num - 1)).astype(jnp.int32)
    right = left + 1
    dist = (pivots[right] - lam_) / (pivots[right] - pivots[left])

    g = jnp.arange(_NCHT, dtype=jnp.int32)
    pair_idx = jnp.stack([left * _NCHT + g, right * _NCHT + g], axis=1)
    dist16 = jnp.full((_LANES,), dist, dtype=jnp.float32)
    const2d = const.reshape(kernel_num * _NCHT, _CHUNK)

    mesh = plsc.VectorSubcoreMesh(core_axis_name="c", subcore_axis_name="s")
    f = pl.kernel(
        _lerp_body,
        out_type=jax.ShapeDtypeStruct((_SIZE,), jnp.float32),
        mesh=mesh,
        scratch_types=[
            pltpu.VMEM((_CHUNKS_PER_W, 2), jnp.int32),
            pltpu.VMEM((2, _CHUNK), jnp.float32),
            pltpu.VMEM((_CHUNK,), jnp.float32),
            pltpu.VMEM((_LANES,), jnp.float32),
            pltpu.SemaphoreType.DMA,
            pltpu.SemaphoreType.DMA,
        ],
    )
    return f(const2d, pair_idx, dist16)


# speedup vs baseline: 1.0835x; 1.0835x over previous
import jax
import jax.numpy as jnp
from jax import lax
from jax.experimental import pallas as pl
from jax.experimental.pallas import tpu as pltpu
from jax.experimental.pallas import tpu_sc as plsc

_NUM_CORES = 2
_NUM_SUBCORES = 16
_NUM_WORKERS = _NUM_CORES * _NUM_SUBCORES
_LANES = 16

_SIZE = 8388608
_CHUNK = 16384
_NCHT = _SIZE // _CHUNK
_PER_W = _SIZE // _NUM_WORKERS
_CHUNKS_PER_W = _PER_W // _CHUNK
_NBUF = 2


def _lerp_body(const2d, pair_idx, dist16, out_hbm,
               idx_v, in0, in1, ob0, ob1, dist_v,
               gsem0, gsem1, ssem0, ssem1):
    c = lax.axis_index("c")
    s = lax.axis_index("s")
    w = s * _NUM_CORES + c
    base_chunk = w * _CHUNKS_PER_W
    out_base = w * _PER_W

    in_bufs = (in0, in1)
    out_bufs = (ob0, ob1)
    gsems = (gsem0, gsem1)
    ssems = (ssem0, ssem1)

    pltpu.sync_copy(pair_idx.at[pl.ds(base_chunk, _CHUNKS_PER_W)], idx_v)
    pltpu.sync_copy(dist16, dist_v)
    dist = dist_v[...]
    omd = 1.0 - dist

    def gather(k, b):
        return pltpu.make_async_copy(
            const2d.at[idx_v.at[k]], in_bufs[b], gsems[b])

    def scatter(k, b):
        return pltpu.make_async_copy(
            out_bufs[b], out_hbm.at[pl.ds(out_base + k * _CHUNK, _CHUNK)],
            ssems[b])

    for b in range(_NBUF):
        gather(b, b).start()

    @pl.loop(0, _CHUNKS_PER_W, step=_NBUF)
    def _(k0):
        for b in range(_NBUF):
            k = k0 + b
            gather(k, b).wait()

            @pl.when(k >= _NBUF)
            def _():
                scatter(k - _NBUF, b).wait()

            ib = in_bufs[b]
            obuf = out_bufs[b]

            @plsc.parallel_loop(0, _CHUNK // _LANES, unroll=8)
            def _(j):
                l = ib[0, pl.ds(j * _LANES, _LANES)]
                r = ib[1, pl.ds(j * _LANES, _LANES)]
                obuf[pl.ds(j * _LANES, _LANES)] = l * dist + r * omd

            @pl.when(k + _NBUF < _CHUNKS_PER_W)
            def _():
                gather(k + _NBUF, b).start()

            scatter(k, b).start()

    for b in range(_NBUF):
        scatter(_CHUNKS_PER_W - _NBUF + b, b).wait()


def kernel(lam, const, pivots):
    kernel_num = const.shape[0]
    lam_ = lam * 0.99999
    left = jnp.floor(lam_ * (kernel_num - 1)).astype(jnp.int32)
    right = left + 1
    dist = (pivots[right] - lam_) / (pivots[right] - pivots[left])

    g = jnp.arange(_NCHT, dtype=jnp.int32)
    pair_idx = jnp.stack([left * _NCHT + g, right * _NCHT + g], axis=1)
    dist16 = jnp.full((_LANES,), dist, dtype=jnp.float32)
    const2d = const.reshape(kernel_num * _NCHT, _CHUNK)

    mesh = plsc.VectorSubcoreMesh(core_axis_name="c", subcore_axis_name="s")
    f = pl.kernel(
        _lerp_body,
        out_type=jax.ShapeDtypeStruct((_SIZE,), jnp.float32),
        mesh=mesh,
        scratch_types=[
            pltpu.VMEM((_CHUNKS_PER_W, 2), jnp.int32),
            pltpu.VMEM((2, _CHUNK), jnp.float32),
            pltpu.VMEM((2, _CHUNK), jnp.float32),
            pltpu.VMEM((_CHUNK,), jnp.float32),
            pltpu.VMEM((_CHUNK,), jnp.float32),
            pltpu.VMEM((_LANES,), jnp.float32),
            pltpu.SemaphoreType.DMA,
            pltpu.SemaphoreType.DMA,
            pltpu.SemaphoreType.DMA,
            pltpu.SemaphoreType.DMA,
        ],
    )
    return f(const2d, pair_idx, dist16)


# speedup vs baseline: 4.1102x; 3.7935x over previous
import jax
import jax.numpy as jnp
from jax import lax
from jax.experimental import pallas as pl
from jax.experimental.pallas import tpu as pltpu
from jax.experimental.pallas import tpu_sc as plsc

_NUM_CORES = 2
_NUM_SUBCORES = 16
_NUM_WORKERS = _NUM_CORES * _NUM_SUBCORES
_LANES = 16

_SIZE = 8388608
_CHUNK = 16384
_PER_W = _SIZE // _NUM_WORKERS
_CHUNKS_PER_W = _PER_W // _CHUNK
_NBUF = 2


def _lerp_body(left16, dist16, const_flat, out_hbm,
               lv, dv, in0, in1, ob0, ob1,
               gsem0, gsem1, ssem0, ssem1):
    c = lax.axis_index("c")
    s = lax.axis_index("s")
    w = s * _NUM_CORES + c

    in_bufs = (in0, in1)
    out_bufs = (ob0, ob1)
    gsems = (gsem0, gsem1)
    ssems = (ssem0, ssem1)

    pltpu.sync_copy(left16, lv)
    pltpu.sync_copy(dist16, dv)
    dist = dv[...]
    omd = 1.0 - dist

    lbase = lv[...][0] * _SIZE + w * _PER_W
    rbase = lbase + _SIZE
    obase = w * _PER_W

    def gathers(k, b):
        ib = in_bufs[b]
        return (
            pltpu.make_async_copy(
                const_flat.at[pl.ds(lbase + k * _CHUNK, _CHUNK)],
                ib.at[0], gsems[b]),
            pltpu.make_async_copy(
                const_flat.at[pl.ds(rbase + k * _CHUNK, _CHUNK)],
                ib.at[1], gsems[b]),
        )

    def scatter(k, b):
        return pltpu.make_async_copy(
            out_bufs[b], out_hbm.at[pl.ds(obase + k * _CHUNK, _CHUNK)],
            ssems[b])

    def start_gathers(k, b):
        gl, gr = gathers(k, b)
        gl.start()
        gr.start()

    def wait_gathers(k, b):
        gl, gr = gathers(k, b)
        gl.wait()
        gr.wait()

    for b in range(_NBUF):
        start_gathers(b, b)

    @pl.loop(0, _CHUNKS_PER_W, step=_NBUF)
    def _(k0):
        for b in range(_NBUF):
            k = k0 + b
            wait_gathers(k, b)

            @pl.when(k >= _NBUF)
            def _():
                scatter(k - _NBUF, b).wait()

            ib = in_bufs[b]
            obuf = out_bufs[b]

            @plsc.parallel_loop(0, _CHUNK // _LANES, unroll=8)
            def _(j):
                l = ib[0, pl.ds(j * _LANES, _LANES)]
                r = ib[1, pl.ds(j * _LANES, _LANES)]
                obuf[pl.ds(j * _LANES, _LANES)] = l * dist + r * omd

            @pl.when(k + _NBUF < _CHUNKS_PER_W)
            def _():
                start_gathers(k + _NBUF, b)

            scatter(k, b).start()

    for b in range(_NBUF):
        scatter(_CHUNKS_PER_W - _NBUF + b, b).wait()


def kernel(lam, const, pivots):
    kernel_num = const.shape[0]
    lam_ = lam * 0.99999
    left = jnp.floor(lam_ * (kernel_num - 1)).astype(jnp.int32)
    right = left + 1
    dist = (pivots[right] - lam_) / (pivots[right] - pivots[left])

    left16 = jnp.full((_LANES,), left, dtype=jnp.int32)
    dist16 = jnp.full((_LANES,), dist, dtype=jnp.float32)
    const_flat = const.reshape(kernel_num * _SIZE)

    mesh = plsc.VectorSubcoreMesh(core_axis_name="c", subcore_axis_name="s")
    f = pl.kernel(
        _lerp_body,
        out_type=jax.ShapeDtypeStruct((_SIZE,), jnp.float32),
        mesh=mesh,
        scratch_types=[
            pltpu.VMEM((_LANES,), jnp.int32),
            pltpu.VMEM((_LANES,), jnp.float32),
            pltpu.VMEM((2, _CHUNK), jnp.float32),
            pltpu.VMEM((2, _CHUNK), jnp.float32),
            pltpu.VMEM((_CHUNK,), jnp.float32),
            pltpu.VMEM((_CHUNK,), jnp.float32),
            pltpu.SemaphoreType.DMA,
            pltpu.SemaphoreType.DMA,
            pltpu.SemaphoreType.DMA,
            pltpu.SemaphoreType.DMA,
        ],
    )
    return f(left16, dist16, const_flat)


# speedup vs baseline: 15.7217x; 3.8250x over previous
import jax
import jax.numpy as jnp
from jax import lax
from jax.experimental import pallas as pl
from jax.experimental.pallas import tpu as pltpu
from jax.experimental.pallas import tpu_sc as plsc

_NUM_CORES = 2
_NUM_SUBCORES = 16
_NUM_WORKERS = _NUM_CORES * _NUM_SUBCORES
_LANES = 16

_SIZE = 8388608
_CHUNK = 16384
_PER_W = _SIZE // _NUM_WORKERS
_CHUNKS_PER_W = _PER_W // _CHUNK
_NBUF = 2


def _lerp_body(left16, dist16, const_hbm, out_hbm,
               lv, dv, in0, in1, ob0, ob1,
               gsem0, gsem1, ssem0, ssem1):
    c = lax.axis_index("c")
    s = lax.axis_index("s")
    w = s * _NUM_CORES + c

    in_bufs = (in0, in1)
    out_bufs = (ob0, ob1)
    gsems = (gsem0, gsem1)
    ssems = (ssem0, ssem1)

    pltpu.sync_copy(left16, lv)
    pltpu.sync_copy(dist16, dv)
    dist = dv[...]
    omd = 1.0 - dist

    lrow = lv[...][0]
    rrow = lrow + 1
    cbase = w * _PER_W
    obase = w * _PER_W

    def gathers(k, b):
        ib = in_bufs[b]
        return (
            pltpu.make_async_copy(
                const_hbm.at[lrow, pl.ds(cbase + k * _CHUNK, _CHUNK)],
                ib.at[0], gsems[b]),
            pltpu.make_async_copy(
                const_hbm.at[rrow, pl.ds(cbase + k * _CHUNK, _CHUNK)],
                ib.at[1], gsems[b]),
        )

    def scatter(k, b):
        return pltpu.make_async_copy(
            out_bufs[b], out_hbm.at[pl.ds(obase + k * _CHUNK, _CHUNK)],
            ssems[b])

    def start_gathers(k, b):
        gl, gr = gathers(k, b)
        gl.start()
        gr.start()

    def wait_gathers(k, b):
        gl, gr = gathers(k, b)
        gl.wait()
        gr.wait()

    for b in range(_NBUF):
        start_gathers(b, b)

    @pl.loop(0, _CHUNKS_PER_W, step=_NBUF)
    def _(k0):
        for b in range(_NBUF):
            k = k0 + b
            wait_gathers(k, b)

            @pl.when(k >= _NBUF)
            def _():
                scatter(k - _NBUF, b).wait()

            ib = in_bufs[b]
            obuf = out_bufs[b]

            @plsc.parallel_loop(0, _CHUNK // _LANES, unroll=8)
            def _(j):
                l = ib[0, pl.ds(j * _LANES, _LANES)]
                r = ib[1, pl.ds(j * _LANES, _LANES)]
                obuf[pl.ds(j * _LANES, _LANES)] = l * dist + r * omd

            @pl.when(k + _NBUF < _CHUNKS_PER_W)
            def _():
                start_gathers(k + _NBUF, b)

            scatter(k, b).start()

    for b in range(_NBUF):
        scatter(_CHUNKS_PER_W - _NBUF + b, b).wait()


def kernel(lam, const, pivots):
    kernel_num = const.shape[0]
    lam_ = lam * 0.99999
    left = jnp.floor(lam_ * (kernel_num - 1)).astype(jnp.int32)
    right = left + 1
    dist = (pivots[right] - lam_) / (pivots[right] - pivots[left])

    left16 = jnp.full((_LANES,), left, dtype=jnp.int32)
    dist16 = jnp.full((_LANES,), dist, dtype=jnp.float32)

    mesh = plsc.VectorSubcoreMesh(core_axis_name="c", subcore_axis_name="s")
    f = pl.kernel(
        _lerp_body,
        out_type=jax.ShapeDtypeStruct((_SIZE,), jnp.float32),
        mesh=mesh,
        scratch_types=[
            pltpu.VMEM((_LANES,), jnp.int32),
            pltpu.VMEM((_LANES,), jnp.float32),
            pltpu.VMEM((2, _CHUNK), jnp.float32),
            pltpu.VMEM((2, _CHUNK), jnp.float32),
            pltpu.VMEM((_CHUNK,), jnp.float32),
            pltpu.VMEM((_CHUNK,), jnp.float32),
            pltpu.SemaphoreType.DMA,
            pltpu.SemaphoreType.DMA,
            pltpu.SemaphoreType.DMA,
            pltpu.SemaphoreType.DMA,
        ],
    )
    return f(left16, dist16, const)
